# Initial kernel scaffold; baseline (speedup 1.0000x reference)
import functools

import jax
import jax.numpy as jnp
from jax import lax
from jax.experimental import pallas as pl
from jax.experimental.pallas import tpu as pltpu
from jax.experimental.pallas import tpu_sc as plsc

N = 10000
E = 320000
NC, NS = 2, 16
NW = NC * NS

MESH = plsc.VectorSubcoreMesh(core_axis_name="c", subcore_axis_name="s")


EPT_DEG = E // NW


@functools.partial(
    pl.kernel,
    out_type=jax.ShapeDtypeStruct((NW, N), jnp.float32),
    mesh=MESH,
    scratch_types=[
        pltpu.VMEM((EPT_DEG,), jnp.int32),
        pltpu.VMEM((N,), jnp.float32),
    ],
)
def _degree_kernel(col_hbm, out_hbm, col_v, acc_v):
    c = lax.axis_index("c")
    s = lax.axis_index("s")
    wid = c * NS + s
    pltpu.sync_copy(col_hbm.at[pl.ds(wid * EPT_DEG, EPT_DEG)], col_v)

    def zero_body(i, _):
        acc_v[pl.ds(i * 16, 16)] = jnp.zeros((16,), jnp.float32)
        return 0

    lax.fori_loop(0, N // 16, zero_body, 0)

    ones = jnp.ones((16,), jnp.float32)

    def hist_body(i, _):
        idx = col_v[pl.ds(i * 16, 16)]
        plsc.addupdate_scatter(acc_v, [idx], ones)
        return 0

    lax.fori_loop(0, EPT_DEG // 16, hist_body, 0)
    pltpu.sync_copy(acc_v, out_hbm.at[wid])


def _dinv_body(parts_ref, out_ref):
    deg = jnp.sum(parts_ref[...], axis=0) + 1.0
    out_ref[...] = lax.rsqrt(deg)[None, :]


def _compute_dinv(parts):
    out = pl.pallas_call(
        _dinv_body,
        out_shape=jax.ShapeDtypeStruct((1, N), jnp.float32),
    )(parts)
    return out.reshape(N, 1)


MB = 1000


def _mm_pre_body(x_ref, w_ref, dinv_ref, out_ref):
    h = jnp.dot(x_ref[...], w_ref[...], preferred_element_type=jnp.float32)
    out_ref[0] = dinv_ref[...] * h


def _mm_pre(x, W, dinv):
    d_in, d_out = W.shape
    dh = d_out // 2
    return pl.pallas_call(
        _mm_pre_body,
        grid=(N // MB, 2),
        in_specs=[
            pl.BlockSpec((MB, d_in), lambda i, c: (i, 0)),
            pl.BlockSpec((d_in, dh), lambda i, c: (0, c)),
            pl.BlockSpec((MB, 1), lambda i, c: (i, 0)),
        ],
        out_specs=pl.BlockSpec((1, MB, dh), lambda i, c: (c, i, 0)),
        out_shape=jax.ShapeDtypeStruct((2, N, dh), jnp.float32),
    )(x, W, dinv)


def _fuse_body(sp_ref, dinv_ref, b_ref, w_ref, out_ref, *, dh_prev):
    dinv = dinv_ref[...]
    a0 = jnp.maximum(dinv * sp_ref[0] + b_ref[0, :dh_prev], 0.0)
    a1 = jnp.maximum(dinv * sp_ref[1] + b_ref[0, dh_prev:], 0.0)
    h = (jnp.dot(a0, w_ref[:dh_prev], preferred_element_type=jnp.float32)
         + jnp.dot(a1, w_ref[dh_prev:], preferred_element_type=jnp.float32))
    out_ref[0] = dinv * h


def _fuse(sp, dinv, b, W):
    dh_prev = sp.shape[2]
    d_in, d_out = W.shape
    dh = d_out // 2
    return pl.pallas_call(
        functools.partial(_fuse_body, dh_prev=dh_prev),
        grid=(N // MB, 2),
        in_specs=[
            pl.BlockSpec((2, MB, dh_prev), lambda i, c: (0, i, 0)),
            pl.BlockSpec((MB, 1), lambda i, c: (i, 0)),
            pl.BlockSpec((1, d_in), lambda i, c: (0, 0)),
            pl.BlockSpec((d_in, dh), lambda i, c: (0, c)),
        ],
        out_specs=pl.BlockSpec((1, MB, dh), lambda i, c: (c, i, 0)),
        out_shape=jax.ShapeDtypeStruct((2, N, dh), jnp.float32),
    )(sp, dinv, b, W)


def _final_body(sp_ref, dinv_ref, b_ref, out_ref):
    z = jnp.concatenate([sp_ref[0], sp_ref[1]], axis=-1)
    z = dinv_ref[...] * z + b_ref[...]
    m = jnp.max(z, axis=1, keepdims=True)
    lse = jnp.log(jnp.sum(jnp.exp(z - m), axis=1, keepdims=True)) + m
    out_ref[...] = z - lse


def _final(sp, dinv, b):
    dh = sp.shape[2]
    d = 2 * dh
    return pl.pallas_call(
        _final_body,
        grid=(N // MB,),
        in_specs=[
            pl.BlockSpec((2, MB, dh), lambda i: (0, i, 0)),
            pl.BlockSpec((MB, 1), lambda i: (i, 0)),
            pl.BlockSpec((1, d), lambda i: (0, 0)),
        ],
        out_specs=pl.BlockSpec((MB, d), lambda i: (i, 0)),
        out_shape=jax.ShapeDtypeStruct((N, d), jnp.float32),
    )(sp, dinv, b)


EPT = E // NS
CB = 80
NCHUNK = EPT // CB
RPT = N // NS


def _make_prop(dh):
    @functools.partial(
        pl.kernel,
        out_type=jax.ShapeDtypeStruct((2 * N, dh), jnp.float32),
        mesh=MESH,
        scratch_types=[
            pltpu.VMEM((CB,), jnp.int32),
            pltpu.VMEM((CB,), jnp.int32),
            pltpu.VMEM((CB, dh), jnp.float32),
            pltpu.VMEM_SHARED((N, dh), jnp.float32),
            pltpu.SemaphoreType.DMA,
        ],
    )
    def prop(p_hbm, row_hbm, col_hbm, out_hbm, row_v, col_v, buf, acc, sem):
        c = lax.axis_index("c")
        s = lax.axis_index("s")
        pltpu.sync_copy(p_hbm.at[pl.ds(c * N + s * RPT, RPT)],
                        acc.at[pl.ds(s * RPT, RPT)])
        plsc.subcore_barrier()

        ebase = s * EPT
        off = jnp.full((16,), c * N, jnp.int32)

        def chunk_body(k, _):
            pltpu.sync_copy(row_hbm.at[pl.ds(ebase + k * CB, CB)], row_v)
            pltpu.sync_copy(col_hbm.at[pl.ds(ebase + k * CB, CB)], col_v)

            def off_body(j, _):
                row_v[pl.ds(j * 16, 16)] = row_v[pl.ds(j * 16, 16)] + off
                return 0

            lax.fori_loop(0, CB // 16, off_body, 0)
            pltpu.async_copy(p_hbm.at[row_v], buf, sem).wait()
            pltpu.sync_copy(buf, acc.at[col_v], add=True)
            return 0

        lax.fori_loop(0, NCHUNK, chunk_body, 0)
        plsc.subcore_barrier()
        pltpu.sync_copy(acc.at[pl.ds(s * RPT, RPT)],
                        out_hbm.at[pl.ds(c * N + s * RPT, RPT)])

    return prop


def _propagate(p, row, col):
    dh = p.shape[2]
    sp = _make_prop(dh)(p.reshape(2 * N, dh), row, col)
    return sp.reshape(2, N, dh)


def kernel(x, edge_index, W1, b1, W2, b2, W3, b3):
    row = edge_index[0].astype(jnp.int32)
    col = edge_index[1].astype(jnp.int32)

    parts = _degree_kernel(col)
    dinv = _compute_dinv(parts)

    p1 = _mm_pre(x, W1, dinv)
    sp1 = _propagate(p1, row, col)
    p2 = _fuse(sp1, dinv, b1.reshape(1, -1), W2)
    sp2 = _propagate(p2, row, col)
    p3 = _fuse(sp2, dinv, b2.reshape(1, -1), W3)
    sp3 = _propagate(p3, row, col)
    return _final(sp3, dinv, b3.reshape(1, -1))


# speedup vs baseline: 8.0686x; 8.0686x over previous
import functools

import jax
import jax.numpy as jnp
from jax import lax
from jax.experimental import pallas as pl
from jax.experimental.pallas import tpu as pltpu
from jax.experimental.pallas import tpu_sc as plsc

N = 10000
NP = 10240
E = 320000
D = 128
NC, NS = 2, 16
NW = NC * NS

MESH = plsc.VectorSubcoreMesh(core_axis_name="c", subcore_axis_name="s")

CB = 80
RPT = NP // NS


@functools.partial(
    pl.kernel,
    out_type=jax.ShapeDtypeStruct((2 * NP, D), jnp.float32),
    mesh=MESH,
    scratch_types=[
        pltpu.VMEM((CB,), jnp.int32),
        pltpu.VMEM((CB,), jnp.int32),
        pltpu.VMEM((CB, D), jnp.float32),
        pltpu.VMEM_SHARED((NP, D), jnp.float32),
        pltpu.SemaphoreType.DMA,
    ],
)
def _prop_edge(p_hbm, z_hbm, row_hbm, col_hbm, out_hbm,
               row_v, col_v, buf, acc, sem):
    c = lax.axis_index("c")
    s = lax.axis_index("s")
    stripe = pl.ds(s * RPT, RPT)

    @pl.when(c == 0)
    def _():
        pltpu.sync_copy(p_hbm.at[stripe], acc.at[stripe])

    @pl.when(c != 0)
    def _():
        pltpu.sync_copy(z_hbm.at[stripe], acc.at[stripe])

    plsc.subcore_barrier()

    ept = E // NW
    ebase = c * (E // NC) + s * ept

    def chunk_body(k, _):
        pltpu.sync_copy(row_hbm.at[pl.ds(ebase + k * CB, CB)], row_v)
        pltpu.sync_copy(col_hbm.at[pl.ds(ebase + k * CB, CB)], col_v)
        pltpu.async_copy(p_hbm.at[row_v], buf, sem).wait()
        pltpu.sync_copy(buf, acc.at[col_v], add=True)
        return 0

    lax.fori_loop(0, ept // CB, chunk_body, 0)
    plsc.subcore_barrier()
    pltpu.sync_copy(acc.at[stripe], out_hbm.at[pl.ds(c * NP + s * RPT, RPT)])


@functools.partial(
    pl.kernel,
    out_type=jax.ShapeDtypeStruct((2 * NP, D), jnp.float32),
    mesh=MESH,
    scratch_types=[
        pltpu.VMEM((CB,), jnp.int32),
        pltpu.VMEM((CB,), jnp.int32),
        pltpu.VMEM((CB, D), jnp.float32),
        pltpu.VMEM_SHARED((NP, D), jnp.float32),
        pltpu.SemaphoreType.DMA,
    ],
)
def _prop_feat(p_hbm, row_hbm, col_hbm, out_hbm, row_v, col_v, buf, acc, sem):
    c = lax.axis_index("c")
    s = lax.axis_index("s")
    stripe = pl.ds(s * RPT, RPT)
    pltpu.sync_copy(p_hbm.at[pl.ds(c * NP + s * RPT, RPT)], acc.at[stripe])
    plsc.subcore_barrier()

    ept = E // NS
    ebase = s * ept
    off = jnp.full((16,), c * NP, jnp.int32)

    def chunk_body(k, _):
        pltpu.sync_copy(row_hbm.at[pl.ds(ebase + k * CB, CB)], row_v)
        pltpu.sync_copy(col_hbm.at[pl.ds(ebase + k * CB, CB)], col_v)

        def off_body(j, _):
            row_v[pl.ds(j * 16, 16)] = row_v[pl.ds(j * 16, 16)] + off
            return 0

        lax.fori_loop(0, CB // 16, off_body, 0)
        pltpu.async_copy(p_hbm.at[row_v], buf, sem).wait()
        pltpu.sync_copy(buf, acc.at[col_v], add=True)
        return 0

    lax.fori_loop(0, ept // CB, chunk_body, 0)
    plsc.subcore_barrier()
    pltpu.sync_copy(acc.at[stripe], out_hbm.at[pl.ds(c * NP + s * RPT, RPT)])


MB = 1024


def _dinv_body(deg_ref, out_ref):
    deg = deg_ref[0, :, :1] + deg_ref[1, :, :1]
    out_ref[...] = lax.rsqrt(deg)


def _compute_dinv(deg_parts):
    return pl.pallas_call(
        _dinv_body,
        grid=(NP // MB,),
        in_specs=[pl.BlockSpec((2, MB, D), lambda i: (0, i, 0))],
        out_specs=pl.BlockSpec((MB, 1), lambda i: (i, 0)),
        out_shape=jax.ShapeDtypeStruct((NP, 1), jnp.float32),
    )(deg_parts)


def _mm1_body(x_ref, w_ref, dinv_ref, out_ref):
    h = jnp.dot(x_ref[...], w_ref[...], preferred_element_type=jnp.float32)
    out_ref[...] = dinv_ref[...] * h


def _mm1(x, W, dinv):
    d_in, d_out = W.shape
    return pl.pallas_call(
        _mm1_body,
        grid=(NP // MB,),
        in_specs=[
            pl.BlockSpec((MB, d_in), lambda i: (i, 0)),
            pl.BlockSpec((d_in, d_out), lambda i: (0, 0)),
            pl.BlockSpec((MB, 1), lambda i: (i, 0)),
        ],
        out_specs=pl.BlockSpec((MB, d_out), lambda i: (i, 0)),
        out_shape=jax.ShapeDtypeStruct((NP, d_out), jnp.float32),
    )(x, W, dinv)


def _fuse12_body(sp_ref, dinv_ref, b_ref, w_ref, out_ref):
    dinv = dinv_ref[...]
    sp = sp_ref[0] + sp_ref[1]
    a = jnp.maximum(dinv * sp + b_ref[...], 0.0)
    h = jnp.dot(a, w_ref[0], preferred_element_type=jnp.float32)
    out_ref[0] = dinv * h


def _fuse12(sp_parts, dinv, b, W):
    d_in, d_out = W.shape
    dh = d_out // 2
    w_split = W.reshape(d_in, 2, dh).transpose(1, 0, 2)
    return pl.pallas_call(
        _fuse12_body,
        grid=(NP // MB, 2),
        in_specs=[
            pl.BlockSpec((2, MB, d_in), lambda i, c: (0, i, 0)),
            pl.BlockSpec((MB, 1), lambda i, c: (i, 0)),
            pl.BlockSpec((1, d_in), lambda i, c: (0, 0)),
            pl.BlockSpec((1, d_in, dh), lambda i, c: (c, 0, 0)),
        ],
        out_specs=pl.BlockSpec((1, MB, dh), lambda i, c: (c, i, 0)),
        out_shape=jax.ShapeDtypeStruct((2, NP, dh), jnp.float32),
    )(sp_parts, dinv, b, w_split)


def _fuse23_body(sp_ref, dinv_ref, b_ref, w_ref, out_ref):
    dinv = dinv_ref[...]
    a0 = jnp.maximum(dinv * sp_ref[0] + b_ref[0, :D], 0.0)
    a1 = jnp.maximum(dinv * sp_ref[1] + b_ref[0, D:], 0.0)
    h = (jnp.dot(a0, w_ref[:D], preferred_element_type=jnp.float32)
         + jnp.dot(a1, w_ref[D:], preferred_element_type=jnp.float32))
    out_ref[...] = dinv * h


def _fuse23(sp_halves, dinv, b, W):
    d_in, d_out = W.shape
    return pl.pallas_call(
        _fuse23_body,
        grid=(NP // MB,),
        in_specs=[
            pl.BlockSpec((2, MB, D), lambda i: (0, i, 0)),
            pl.BlockSpec((MB, 1), lambda i: (i, 0)),
            pl.BlockSpec((1, d_in), lambda i: (0, 0)),
            pl.BlockSpec((d_in, d_out), lambda i: (0, 0)),
        ],
        out_specs=pl.BlockSpec((MB, d_out), lambda i: (i, 0)),
        out_shape=jax.ShapeDtypeStruct((NP, d_out), jnp.float32),
    )(sp_halves, dinv, b, W)


def _final_body(sp_ref, dinv_ref, b_ref, out_ref):
    z = sp_ref[0] + sp_ref[1]
    z = dinv_ref[...] * z + b_ref[...]
    m = jnp.max(z, axis=1, keepdims=True)
    lse = jnp.log(jnp.sum(jnp.exp(z - m), axis=1, keepdims=True)) + m
    out_ref[...] = z - lse


def _final(sp_parts, dinv, b):
    return pl.pallas_call(
        _final_body,
        grid=(NP // MB,),
        in_specs=[
            pl.BlockSpec((2, MB, D), lambda i: (0, i, 0)),
            pl.BlockSpec((MB, 1), lambda i: (i, 0)),
            pl.BlockSpec((1, D), lambda i: (0, 0)),
        ],
        out_specs=pl.BlockSpec((MB, D), lambda i: (i, 0)),
        out_shape=jax.ShapeDtypeStruct((NP, D), jnp.float32),
    )(sp_parts, dinv, b)


def kernel(x, edge_index, W1, b1, W2, b2, W3, b3):
    row = edge_index[0].astype(jnp.int32)
    col = edge_index[1].astype(jnp.int32)

    zeros = jnp.zeros((NP, D), jnp.float32)
    ones = jnp.ones((NP, D), jnp.float32)

    deg_parts = _prop_edge(ones, zeros, row, col).reshape(2, NP, D)
    dinv = _compute_dinv(deg_parts)

    x_pad = jnp.zeros((NP, x.shape[1]), jnp.float32).at[:N].set(x)
    p1 = _mm1(x_pad, W1, dinv)
    sp1 = _prop_edge(p1, zeros, row, col).reshape(2, NP, D)
    p2 = _fuse12(sp1, dinv, b1.reshape(1, -1), W2)
    sp2 = _prop_feat(p2.reshape(2 * NP, D), row, col).reshape(2, NP, D)
    p3 = _fuse23(sp2, dinv, b2.reshape(1, -1), W3)
    sp3 = _prop_edge(p3, zeros, row, col).reshape(2, NP, D)
    return _final(sp3, dinv, b3.reshape(1, -1))[:N]


# speedup vs baseline: 16.0418x; 1.9882x over previous
import functools

import jax
import jax.numpy as jnp
from jax import lax
from jax.experimental import pallas as pl
from jax.experimental.pallas import tpu as pltpu
from jax.experimental.pallas import tpu_sc as plsc

N = 10000
NP = 10240
E = 320000
D = 128
NC, NS = 2, 16
NW = NC * NS

MESH = plsc.VectorSubcoreMesh(core_axis_name="c", subcore_axis_name="s")

CB = 128
EPT_E = E // NW
EPT_F = E // NS
NCH_E = 80
CBP = 80
NCHP_E = EPT_E // CBP
NCHP_F = EPT_F // CBP
RPT = NP // NS


NBUF = 4


def _pipe_loop(p_hbm, acc, row_hbm, col_hbm, rbase, cbase,
               row_v, col_v, buf, gsem, ssem, nch):
    k0 = nch % NBUF
    for k in range(k0):
        pltpu.sync_copy(row_hbm.at[pl.ds(rbase + k * CBP, CBP)], row_v[0])
        pltpu.sync_copy(col_hbm.at[pl.ds(cbase + k * CBP, CBP)], col_v[0])
        pltpu.async_copy(p_hbm.at[row_v[0]], buf[0], gsem[0]).wait()
        pltpu.sync_copy(buf[0], acc.at[col_v[0]], add=True)

    for b in range(NBUF):
        pltpu.sync_copy(row_hbm.at[pl.ds(rbase + (k0 + b) * CBP, CBP)],
                        row_v[b])
        pltpu.sync_copy(col_hbm.at[pl.ds(cbase + (k0 + b) * CBP, CBP)],
                        col_v[b])
        pltpu.async_copy(p_hbm.at[row_v[b]], buf[b], gsem[b])

    def group_body(t, _):
        k = k0 + NBUF * t
        for b in range(NBUF):
            pltpu.make_async_copy(p_hbm.at[row_v[b]], buf[b], gsem[b]).wait()
            pltpu.async_copy(buf[b], acc.at[col_v[b]], ssem[b], add=True)
        for b in range(NBUF):
            @pl.when(k + b + NBUF < nch)
            def _():
                pltpu.make_async_copy(buf[b], acc.at[col_v[b]],
                                      ssem[b]).wait()
                pltpu.sync_copy(
                    row_hbm.at[pl.ds(rbase + (k + b + NBUF) * CBP, CBP)],
                    row_v[b])
                pltpu.sync_copy(
                    col_hbm.at[pl.ds(cbase + (k + b + NBUF) * CBP, CBP)],
                    col_v[b])
                pltpu.async_copy(p_hbm.at[row_v[b]], buf[b], gsem[b])
        return 0

    lax.fori_loop(0, (nch - k0) // NBUF, group_body, 0)
    for b in range(NBUF):
        pltpu.make_async_copy(buf[b], acc.at[col_v[b]], ssem[b]).wait()


_PROP_SCRATCH = (
    [pltpu.VMEM((CBP,), jnp.int32)] * (2 * NBUF)
    + [pltpu.VMEM((CBP, D), jnp.float32)] * NBUF
    + [pltpu.VMEM_SHARED((NP, D), jnp.float32)]
    + [pltpu.SemaphoreType.DMA] * (2 * NBUF)
)


@functools.partial(
    pl.kernel,
    out_type=jax.ShapeDtypeStruct((2 * NP, D), jnp.float32),
    mesh=MESH,
    scratch_types=_PROP_SCRATCH,
)
def _prop_edge(p_hbm, z_hbm, row_hbm, col_hbm, out_hbm,
               r0, r1, r2, r3, c0, c1, c2, c3, b0, b1, b2, b3, acc,
               g0, g1, g2, g3, s0, s1, s2, s3):
    c = lax.axis_index("c")
    s = lax.axis_index("s")
    wid = c * NS + s
    stripe = pl.ds(s * RPT, RPT)

    @pl.when(c == 0)
    def _():
        pltpu.sync_copy(p_hbm.at[stripe], acc.at[stripe])

    @pl.when(c != 0)
    def _():
        pltpu.sync_copy(z_hbm.at[stripe], acc.at[stripe])

    plsc.subcore_barrier()

    ebase = wid * EPT_E
    _pipe_loop(p_hbm, acc, row_hbm, col_hbm, ebase, ebase,
               [r0, r1, r2, r3], [c0, c1, c2, c3], [b0, b1, b2, b3],
               [g0, g1, g2, g3], [s0, s1, s2, s3], NCHP_E)

    plsc.subcore_barrier()
    pltpu.sync_copy(acc.at[stripe], out_hbm.at[pl.ds(c * NP + s * RPT, RPT)])


@functools.partial(
    pl.kernel,
    out_type=jax.ShapeDtypeStruct((2 * NP, D), jnp.float32),
    mesh=MESH,
    scratch_types=_PROP_SCRATCH,
)
def _prop_feat(p_hbm, row_hbm, col_hbm, out_hbm,
               r0, r1, r2, r3, c0, c1, c2, c3, b0, b1, b2, b3, acc,
               g0, g1, g2, g3, s0, s1, s2, s3):
    c = lax.axis_index("c")
    s = lax.axis_index("s")
    wid = c * NS + s
    stripe = pl.ds(s * RPT, RPT)
    pltpu.sync_copy(p_hbm.at[pl.ds(c * NP + s * RPT, RPT)], acc.at[stripe])
    plsc.subcore_barrier()

    rbase = wid * EPT_F
    cbase = s * EPT_F
    _pipe_loop(p_hbm, acc, row_hbm, col_hbm, rbase, cbase,
               [r0, r1, r2, r3], [c0, c1, c2, c3], [b0, b1, b2, b3],
               [g0, g1, g2, g3], [s0, s1, s2, s3], NCHP_F)

    plsc.subcore_barrier()
    pltpu.sync_copy(acc.at[stripe], out_hbm.at[pl.ds(c * NP + s * RPT, RPT)])


@functools.partial(
    pl.kernel,
    out_type=jax.ShapeDtypeStruct((2 * NP, D), jnp.float32),
    mesh=MESH,
    scratch_types=[
        pltpu.VMEM((CB,), jnp.int32),
        pltpu.VMEM((CB,), jnp.int32),
        pltpu.VMEM((CB, D), jnp.float32),
        pltpu.VMEM_SHARED((NP, D), jnp.float32),
        pltpu.SemaphoreType.DMA,
        pltpu.SemaphoreType.DMA,
    ],
)
def _deg_kernel(col_hbm, ones_hbm, z_hbm, out_hbm,
                col_v0, col_v1, ones_v, acc, sem0, sem1):
    c = lax.axis_index("c")
    s = lax.axis_index("s")
    wid = c * NS + s
    stripe = pl.ds(s * RPT, RPT)
    pltpu.sync_copy(z_hbm.at[stripe], acc.at[stripe])
    pltpu.sync_copy(ones_hbm, ones_v)
    plsc.subcore_barrier()

    ebase = wid * (NCH_E * CB)

    def pair_body(t, _):
        k0 = 2 * t
        pltpu.sync_copy(col_hbm.at[pl.ds(ebase + (k0 + 1) * CB, CB)], col_v1)
        pltpu.async_copy(ones_v, acc.at[col_v1], sem1, add=True)
        pltpu.make_async_copy(ones_v, acc.at[col_v0], sem0).wait()

        @pl.when(t + 1 < NCH_E // 2)
        def _():
            pltpu.sync_copy(col_hbm.at[pl.ds(ebase + (k0 + 2) * CB, CB)],
                            col_v0)
            pltpu.async_copy(ones_v, acc.at[col_v0], sem0, add=True)

        pltpu.make_async_copy(ones_v, acc.at[col_v1], sem1).wait()
        return 0

    pltpu.sync_copy(col_hbm.at[pl.ds(ebase, CB)], col_v0)
    pltpu.async_copy(ones_v, acc.at[col_v0], sem0, add=True)
    lax.fori_loop(0, NCH_E // 2, pair_body, 0)

    plsc.subcore_barrier()
    pltpu.sync_copy(acc.at[stripe], out_hbm.at[pl.ds(c * NP + s * RPT, RPT)])


MB = 1024


def _mm1_body(x_ref, w_ref, out_ref):
    out_ref[...] = jnp.dot(x_ref[...], w_ref[...],
                           preferred_element_type=jnp.float32)


def _mm1(x, W):
    d_in, d_out = W.shape
    return pl.pallas_call(
        _mm1_body,
        grid=(NP // MB,),
        in_specs=[
            pl.BlockSpec((MB, d_in), lambda i: (i, 0)),
            pl.BlockSpec((d_in, d_out), lambda i: (0, 0)),
        ],
        out_specs=pl.BlockSpec((MB, d_out), lambda i: (i, 0)),
        out_shape=jax.ShapeDtypeStruct((NP, d_out), jnp.float32),
    )(x, W)


def _dinv_p1_body(deg_ref, h1_ref, dinv_ref, p1_ref):
    deg = deg_ref[0, :, :1] + deg_ref[1, :, :1] + 1.0
    dinv = lax.rsqrt(deg)
    dinv_ref[...] = dinv
    p1_ref[...] = dinv * h1_ref[...]


def _dinv_p1(deg_parts, h1):
    return pl.pallas_call(
        _dinv_p1_body,
        grid=(NP // MB,),
        in_specs=[
            pl.BlockSpec((2, MB, D), lambda i: (0, i, 0)),
            pl.BlockSpec((MB, D), lambda i: (i, 0)),
        ],
        out_specs=[
            pl.BlockSpec((MB, 1), lambda i: (i, 0)),
            pl.BlockSpec((MB, D), lambda i: (i, 0)),
        ],
        out_shape=[
            jax.ShapeDtypeStruct((NP, 1), jnp.float32),
            jax.ShapeDtypeStruct((NP, D), jnp.float32),
        ],
    )(deg_parts, h1)


def _fuse12_body(sp_ref, dinv_ref, b_ref, w_ref, out_ref):
    dinv = dinv_ref[...]
    sp = sp_ref[0] + sp_ref[1]
    a = jnp.maximum(dinv * sp + b_ref[...], 0.0)
    h = jnp.dot(a, w_ref[0], preferred_element_type=jnp.float32)
    out_ref[0] = dinv * h


def _fuse12(sp_parts, dinv, b, W):
    d_in, d_out = W.shape
    dh = d_out // 2
    w_split = W.reshape(d_in, 2, dh).transpose(1, 0, 2)
    return pl.pallas_call(
        _fuse12_body,
        grid=(NP // MB, 2),
        in_specs=[
            pl.BlockSpec((2, MB, d_in), lambda i, c: (0, i, 0)),
            pl.BlockSpec((MB, 1), lambda i, c: (i, 0)),
            pl.BlockSpec((1, d_in), lambda i, c: (0, 0)),
            pl.BlockSpec((1, d_in, dh), lambda i, c: (c, 0, 0)),
        ],
        out_specs=pl.BlockSpec((1, MB, dh), lambda i, c: (c, i, 0)),
        out_shape=jax.ShapeDtypeStruct((2, NP, dh), jnp.float32),
    )(sp_parts, dinv, b, w_split)


def _fuse23_body(sp_ref, dinv_ref, b_ref, w_ref, out_ref):
    dinv = dinv_ref[...]
    a0 = jnp.maximum(dinv * sp_ref[0] + b_ref[0, :D], 0.0)
    a1 = jnp.maximum(dinv * sp_ref[1] + b_ref[0, D:], 0.0)
    h = (jnp.dot(a0, w_ref[:D], preferred_element_type=jnp.float32)
         + jnp.dot(a1, w_ref[D:], preferred_element_type=jnp.float32))
    out_ref[...] = dinv * h


def _fuse23(sp_halves, dinv, b, W):
    d_in, d_out = W.shape
    return pl.pallas_call(
        _fuse23_body,
        grid=(NP // MB,),
        in_specs=[
            pl.BlockSpec((2, MB, D), lambda i: (0, i, 0)),
            pl.BlockSpec((MB, 1), lambda i: (i, 0)),
            pl.BlockSpec((1, d_in), lambda i: (0, 0)),
            pl.BlockSpec((d_in, d_out), lambda i: (0, 0)),
        ],
        out_specs=pl.BlockSpec((MB, d_out), lambda i: (i, 0)),
        out_shape=jax.ShapeDtypeStruct((NP, d_out), jnp.float32),
    )(sp_halves, dinv, b, W)


def _final_body(sp_ref, dinv_ref, b_ref, out_ref):
    z = sp_ref[0] + sp_ref[1]
    z = dinv_ref[...] * z + b_ref[...]
    m = jnp.max(z, axis=1, keepdims=True)
    lse = jnp.log(jnp.sum(jnp.exp(z - m), axis=1, keepdims=True)) + m
    out_ref[...] = z - lse


def _final(sp_parts, dinv, b):
    return pl.pallas_call(
        _final_body,
        grid=(NP // MB,),
        in_specs=[
            pl.BlockSpec((2, MB, D), lambda i: (0, i, 0)),
            pl.BlockSpec((MB, 1), lambda i: (i, 0)),
            pl.BlockSpec((1, D), lambda i: (0, 0)),
        ],
        out_specs=pl.BlockSpec((MB, D), lambda i: (i, 0)),
        out_shape=jax.ShapeDtypeStruct((NP, D), jnp.float32),
    )(sp_parts, dinv, b)


def kernel(x, edge_index, W1, b1, W2, b2, W3, b3):
    row = edge_index[0].astype(jnp.int32)
    col = edge_index[1].astype(jnp.int32)

    row_e1 = row
    col_e1 = col
    row_f1 = jnp.stack([row, row + NP]).reshape(-1)
    col_f1 = col
    pad_e = NCH_E * CB - EPT_E
    pe = N + (jnp.arange(pad_e, dtype=jnp.int32) % (NP - N))
    col_deg = jnp.concatenate(
        [col.reshape(NW, EPT_E), jnp.broadcast_to(pe, (NW, pad_e))],
        axis=1).reshape(-1)

    zeros = jnp.zeros((NP, D), jnp.float32)
    ones_cb = jnp.ones((CB, D), jnp.float32)

    deg_parts = _deg_kernel(col_deg, ones_cb, zeros).reshape(2, NP, D)
    x_pad = jnp.zeros((NP, x.shape[1]), jnp.float32).at[:N].set(x)
    h1 = _mm1(x_pad, W1)
    dinv, p1 = _dinv_p1(deg_parts, h1)

    sp1 = _prop_edge(p1, zeros, row_e1, col_e1).reshape(2, NP, D)
    p2 = _fuse12(sp1, dinv, b1.reshape(1, -1), W2)
    sp2 = _prop_feat(p2.reshape(2 * NP, D), row_f1, col_f1).reshape(2, NP, D)
    p3 = _fuse23(sp2, dinv, b2.reshape(1, -1), W3)
    sp3 = _prop_edge(p3, zeros, row_e1, col_e1).reshape(2, NP, D)
    return _final(sp3, dinv, b3.reshape(1, -1))[:N]
